# baseline (speedup 1.0000x reference)
import jax
import jax.numpy as jnp
from jax.experimental import pallas as pl
from jax.experimental.pallas import tpu as pltpu

_INV_SQRT2 = 0.7071067811865476
_NCH = 4


def _make_moe_body(D, H, E):
    CH = H // _NCH
    assert CH == D

    def _moe_body(c_ref, rwt_ref, rb_ref, x_ref, w1_hbm, w2_hbm, b1_hbm,
                  b2_hbm, out_ref, w1a, w1b, w2a, w2b, b1s, b2s, stage,
                  esmem, wsmem, sems, bsem):
        i = pl.program_id(0)

        def w1_chunk(e, j):
            return w1_hbm.at[e, :, pl.ds(j * CH, CH)]

        def w2_chunk(e, j):
            return w2_hbm.at[e, pl.ds(j * CH, CH), :]

        def start_slab(srcfn, e):
            for j in range(_NCH):
                pltpu.make_async_copy(srcfn(e, j), stage.at[j],
                                      sems.at[j]).start()

        def land_w1(e, dst):
            for j in range(_NCH):
                pltpu.make_async_copy(w1_chunk(e, j), stage.at[j],
                                      sems.at[j]).wait()
                dst[:, pl.ds(j * CH, CH)] = stage[j].astype(jnp.bfloat16)

        def land_w2(e, dst):
            for j in range(_NCH):
                pltpu.make_async_copy(w2_chunk(e, j), stage.at[j],
                                      sems.at[j]).wait()
                dst[pl.ds(j * CH, CH), :] = stage[j].astype(jnp.bfloat16)

        @pl.when(i == 0)
        def _route_and_fetch_w1a():
            c_mean = jnp.mean(c_ref[...], axis=0, keepdims=True)
            logits = jnp.dot(
                c_mean.astype(jnp.bfloat16),
                rwt_ref[...].astype(jnp.bfloat16),
                preferred_element_type=jnp.float32,
            ) + rb_ref[...]
            lane = jax.lax.broadcasted_iota(jnp.int32, logits.shape, 1)
            m1 = jnp.max(logits)
            e0 = jnp.min(jnp.where(logits == m1, lane, E))
            masked = jnp.where(lane == e0, -jnp.inf, logits)
            m2 = jnp.max(masked)
            e1 = jnp.min(jnp.where(masked == m2, lane, E))
            esmem[0] = e0
            esmem[1] = e1
            t = jnp.exp(m2 - m1)
            wsmem[0] = 1.0 / (1.0 + t)
            wsmem[1] = t / (1.0 + t)
            for c in (
                pltpu.make_async_copy(b1_hbm.at[e0], b1s.at[0], bsem),
                pltpu.make_async_copy(b1_hbm.at[e1], b1s.at[1], bsem),
                pltpu.make_async_copy(b2_hbm.at[e0], b2s.at[0], bsem),
                pltpu.make_async_copy(b2_hbm.at[e1], b2s.at[1], bsem),
            ):
                c.start()
            start_slab(w1_chunk, e0)
            land_w1(e0, w1a)
            start_slab(w2_chunk, e0)
            for c in (
                pltpu.make_async_copy(b1_hbm.at[e0], b1s.at[0], bsem),
                pltpu.make_async_copy(b1_hbm.at[e1], b1s.at[1], bsem),
                pltpu.make_async_copy(b2_hbm.at[e0], b2s.at[0], bsem),
                pltpu.make_async_copy(b2_hbm.at[e1], b2s.at[1], bsem),
            ):
                c.wait()

        xv = x_ref[...].astype(jnp.bfloat16)
        w0 = wsmem[0]
        w1 = wsmem[1]

        ha = jnp.dot(xv, w1a[...], preferred_element_type=jnp.float32)

        @pl.when(i == 0)
        def _land_w2a_start_w1b():
            land_w2(esmem[0], w2a)
            start_slab(w1_chunk, esmem[1])

        ha = ha + b1s[0]
        ha = 0.5 * ha * (1.0 + jax.lax.erf(ha * _INV_SQRT2))
        oa = jnp.dot(ha.astype(jnp.bfloat16), w2a[...],
                     preferred_element_type=jnp.float32) + b2s[0]

        @pl.when(i == 0)
        def _land_w1b_start_w2b():
            land_w1(esmem[1], w1b)
            start_slab(w2_chunk, esmem[1])

        hb = jnp.dot(xv, w1b[...], preferred_element_type=jnp.float32)

        @pl.when(i == 0)
        def _land_w2b():
            land_w2(esmem[1], w2b)

        hb = hb + b1s[1]
        hb = 0.5 * hb * (1.0 + jax.lax.erf(hb * _INV_SQRT2))
        ob = jnp.dot(hb.astype(jnp.bfloat16), w2b[...],
                     preferred_element_type=jnp.float32) + b2s[1]

        out_ref[...] = oa * w0 + ob * w1

    return _moe_body


@jax.jit
def kernel(x, c_states, router_W, router_b, W1, b1, W2, b2):
    B, T, D = x.shape
    E, _, H = W1.shape
    N_CELLS, C = c_states.shape
    M = B * T
    BM = 1024

    x2 = x.reshape(M, D)
    out = pl.pallas_call(
        _make_moe_body(D, H, E),
        grid=(M // BM,),
        in_specs=[
            pl.BlockSpec((N_CELLS, C), lambda i: (0, 0)),
            pl.BlockSpec((C, E), lambda i: (0, 0)),
            pl.BlockSpec((1, E), lambda i: (0, 0)),
            pl.BlockSpec((BM, D), lambda i: (i, 0)),
            pl.BlockSpec(memory_space=pltpu.MemorySpace.HBM),
            pl.BlockSpec(memory_space=pltpu.MemorySpace.HBM),
            pl.BlockSpec(memory_space=pltpu.MemorySpace.HBM),
            pl.BlockSpec(memory_space=pltpu.MemorySpace.HBM),
        ],
        out_specs=pl.BlockSpec((BM, D), lambda i: (i, 0)),
        out_shape=jax.ShapeDtypeStruct((M, D), jnp.float32),
        scratch_shapes=[
            pltpu.VMEM((D, H), jnp.bfloat16),
            pltpu.VMEM((D, H), jnp.bfloat16),
            pltpu.VMEM((H, D), jnp.bfloat16),
            pltpu.VMEM((H, D), jnp.bfloat16),
            pltpu.VMEM((2, 1, H), jnp.float32),
            pltpu.VMEM((2, 1, D), jnp.float32),
            pltpu.VMEM((_NCH, D, H // _NCH), jnp.float32),
            pltpu.SMEM((2,), jnp.int32),
            pltpu.SMEM((2,), jnp.float32),
            pltpu.SemaphoreType.DMA((_NCH,)),
            pltpu.SemaphoreType.DMA,
        ],
        compiler_params=pltpu.CompilerParams(
            dimension_semantics=("arbitrary",),
        ),
    )(c_states, router_W.T, router_b.reshape(1, E), x2, W1, W2,
      b1.reshape(E, 1, H), b2.reshape(E, 1, D))
    return out.reshape(B, T, D)


# speedup vs baseline: 1.0299x; 1.0299x over previous
import jax
import jax.numpy as jnp
from jax.experimental import pallas as pl
from jax.experimental.pallas import tpu as pltpu

_INV_SQRT2 = 0.7071067811865476


def _make_moe_body(D, H, E, NCH):
    CH = H // NCH

    def _moe_body(c_ref, rwt_ref, rb_ref, x_ref, w1_hbm, w2_hbm, b1_hbm,
                  b2_hbm, out_ref, w1a, w1b, w2a, w2b, b1s, b2s, stage,
                  wsmem, sems, bsem):
        i = pl.program_id(0)

        @pl.when(i == 0)
        def _prologue():
            c_mean = jnp.mean(c_ref[...], axis=0, keepdims=True)
            logits = jnp.dot(
                c_mean.astype(jnp.bfloat16),
                rwt_ref[...].astype(jnp.bfloat16),
                preferred_element_type=jnp.float32,
            ) + rb_ref[...]
            lane = jax.lax.broadcasted_iota(jnp.int32, logits.shape, 1)
            m1 = jnp.max(logits)
            e0 = jnp.min(jnp.where(logits == m1, lane, E))
            masked = jnp.where(lane == e0, -jnp.inf, logits)
            m2 = jnp.max(masked)
            e1 = jnp.min(jnp.where(masked == m2, lane, E))
            t = jnp.exp(m2 - m1)
            wsmem[0] = 1.0 / (1.0 + t)
            wsmem[1] = t / (1.0 + t)
            bias_copies = [
                pltpu.make_async_copy(b1_hbm.at[e0], b1s.at[0], bsem),
                pltpu.make_async_copy(b1_hbm.at[e1], b1s.at[1], bsem),
                pltpu.make_async_copy(b2_hbm.at[e0], b2s.at[0], bsem),
                pltpu.make_async_copy(b2_hbm.at[e1], b2s.at[1], bsem),
            ]
            for c in bias_copies:
                c.start()
            chunks = []
            for e, w1d, w2d in ((e0, w1a, w2a), (e1, w1b, w2b)):
                for c in range(NCH):
                    sl = pl.ds(c * CH, CH)
                    chunks.append((w1_hbm.at[e, :, sl], w1d,
                                   (slice(None), sl)))
                for c in range(NCH):
                    sl = pl.ds(c * CH, CH)
                    chunks.append((w2_hbm.at[e, sl, :], w2d,
                                   (sl, slice(None))))
            copies = []
            for k, (src, _, _) in enumerate(chunks):
                copies.append(
                    pltpu.make_async_copy(src, stage.at[k % 2], sems.at[k % 2]))
            for k, (_, dst, dsl) in enumerate(chunks):
                copies[k].start()
                if k > 0:
                    copies[k - 1].wait()
                    _, pdst, pdsl = chunks[k - 1]
                    pdst[pdsl] = stage[(k - 1) % 2].astype(jnp.bfloat16)
            copies[-1].wait()
            _, ldst, ldsl = chunks[-1]
            ldst[ldsl] = stage[(len(chunks) - 1) % 2].astype(jnp.bfloat16)
            for c in bias_copies:
                c.wait()

        xv = x_ref[...].astype(jnp.bfloat16)

        def expert(w1_s, b1_i, w2_s, b2_i):
            h = jnp.dot(xv, w1_s[...], preferred_element_type=jnp.float32)
            h = h + b1s[b1_i]
            h = 0.5 * h * (1.0 + jax.lax.erf(h * _INV_SQRT2))
            return jnp.dot(h.astype(jnp.bfloat16), w2_s[...],
                           preferred_element_type=jnp.float32) + b2s[b2_i]

        w0 = wsmem[0]
        w1 = wsmem[1]
        out_ref[...] = (expert(w1a, 0, w2a, 0) * w0
                        + expert(w1b, 1, w2b, 1) * w1)

    return _moe_body


@jax.jit
def kernel(x, c_states, router_W, router_b, W1, b1, W2, b2):
    B, T, D = x.shape
    E, _, H = W1.shape
    N_CELLS, C = c_states.shape
    M = B * T
    BM = 1024
    NCH = 4

    x2 = x.reshape(M, D)
    out = pl.pallas_call(
        _make_moe_body(D, H, E, NCH),
        grid=(M // BM,),
        in_specs=[
            pl.BlockSpec((N_CELLS, C), lambda i: (0, 0)),
            pl.BlockSpec((C, E), lambda i: (0, 0)),
            pl.BlockSpec((1, E), lambda i: (0, 0)),
            pl.BlockSpec((BM, D), lambda i: (i, 0)),
            pl.BlockSpec(memory_space=pltpu.MemorySpace.HBM),
            pl.BlockSpec(memory_space=pltpu.MemorySpace.HBM),
            pl.BlockSpec(memory_space=pltpu.MemorySpace.HBM),
            pl.BlockSpec(memory_space=pltpu.MemorySpace.HBM),
        ],
        out_specs=pl.BlockSpec((BM, D), lambda i: (i, 0)),
        out_shape=jax.ShapeDtypeStruct((M, D), jnp.float32),
        scratch_shapes=[
            pltpu.VMEM((D, H), jnp.bfloat16),
            pltpu.VMEM((D, H), jnp.bfloat16),
            pltpu.VMEM((H, D), jnp.bfloat16),
            pltpu.VMEM((H, D), jnp.bfloat16),
            pltpu.VMEM((2, 1, H), jnp.float32),
            pltpu.VMEM((2, 1, D), jnp.float32),
            pltpu.VMEM((2, D, H // NCH), jnp.float32),
            pltpu.SMEM((2,), jnp.float32),
            pltpu.SemaphoreType.DMA((2,)),
            pltpu.SemaphoreType.DMA,
        ],
        compiler_params=pltpu.CompilerParams(
            dimension_semantics=("arbitrary",),
        ),
    )(c_states, router_W.T, router_b.reshape(1, E), x2, W1, W2,
      b1.reshape(E, 1, H), b2.reshape(E, 1, D))
    return out.reshape(B, T, D)
